# baseline (speedup 1.0000x reference)
import functools

import jax
import jax.numpy as jnp
from jax import lax
from jax.experimental import pallas as pl
from jax.experimental.pallas import tpu as pltpu
from jax.experimental.pallas import tpu_sc as plsc

_U = 10000
_E = 320000
_DG = 64
_C = 256
_NCH = _E // _C
_NW = 32
_IDXK = _C // 128
_ROWS_PER_SUB = 624
_TAIL_ROW0 = 16 * _ROWS_PER_SUB
_TAIL_ROWS = _U - _TAIL_ROW0


def _mm_body(x_ref, w_ref, b_ref, o_ref):
    o_ref[...] = (
        jnp.dot(x_ref[...], w_ref[...], preferred_element_type=jnp.float32)
        + b_ref[...]
    )


def _mm(x, w, b, br):
    m, k = x.shape
    n = w.shape[1]
    return pl.pallas_call(
        _mm_body,
        grid=(m // br,),
        in_specs=[
            pl.BlockSpec((br, k), lambda i: (i, 0)),
            pl.BlockSpec((k, n), lambda i: (0, 0)),
            pl.BlockSpec((1, n), lambda i: (0, 0)),
        ],
        out_specs=pl.BlockSpec((br, n), lambda i: (i, 0)),
        out_shape=jax.ShapeDtypeStruct((m, n), jnp.float32),
    )(x, w, b.reshape(1, n))


def _eg_body(xa_ref, xb_ref, w_ref, o_ref):
    o_ref[...] = jnp.concatenate(
        [jnp.dot(xa_ref[...], w_ref[...], preferred_element_type=jnp.float32),
         jnp.dot(xb_ref[...], w_ref[...], preferred_element_type=jnp.float32)],
        axis=1,
    )


def _eg_mm(x, w, br):
    m2, k = x.shape
    m = m2 // 2
    return pl.pallas_call(
        _eg_body,
        grid=(m // br,),
        in_specs=[
            pl.BlockSpec((br, k), lambda i: (2 * i, 0)),
            pl.BlockSpec((br, k), lambda i: (2 * i + 1, 0)),
            pl.BlockSpec((k, _DG), lambda i: (0, 0)),
        ],
        out_specs=pl.BlockSpec((br, 2 * _DG), lambda i: (i, 0)),
        out_shape=jax.ShapeDtypeStruct((m, 2 * _DG), jnp.float32),
    )(x, x, w)


def _f_body(u_ref, a0_ref, a1_ref, wu_ref, wa_ref, b_ref, o_ref):
    acc = jnp.dot(u_ref[...], wu_ref[...], preferred_element_type=jnp.float32)
    acc = acc + jnp.dot(
        a0_ref[...] + a1_ref[...], wa_ref[...],
        preferred_element_type=jnp.float32,
    )
    o_ref[...] = jnp.maximum(acc + b_ref[...], 0.0)


def _f_mm(u, a0, a1, wu, wa, b, br):
    m, k = u.shape
    ka = a0.shape[1]
    n = wu.shape[1]
    return pl.pallas_call(
        _f_body,
        grid=(m // br,),
        in_specs=[
            pl.BlockSpec((br, k), lambda i: (i, 0)),
            pl.BlockSpec((br, ka), lambda i: (i, 0)),
            pl.BlockSpec((br, ka), lambda i: (i, 0)),
            pl.BlockSpec((k, n), lambda i: (0, 0)),
            pl.BlockSpec((ka, n), lambda i: (0, 0)),
            pl.BlockSpec((1, n), lambda i: (0, 0)),
        ],
        out_specs=pl.BlockSpec((br, n), lambda i: (i, 0)),
        out_shape=jax.ShapeDtypeStruct((m, n), jnp.float32),
    )(u, a0, a1, wu, wa, b.reshape(1, n))


def _sc_edge_body(ug, vg, eg3, dst3, src3, zeros_hbm, out, idx_d, idx_s,
                  eg_in, bu, bv, agg_sh, sem):
    cid = lax.axis_index("c")
    sid = lax.axis_index("s")
    wid = sid * 2 + cid

    row0 = sid * _ROWS_PER_SUB
    pltpu.sync_copy(
        zeros_hbm.at[pl.ds(row0, _ROWS_PER_SUB)],
        agg_sh.at[pl.ds(row0, _ROWS_PER_SUB)],
    )

    @pl.when(sid == 15)
    def _():
        pltpu.sync_copy(
            zeros_hbm.at[pl.ds(_TAIL_ROW0, _TAIL_ROWS)],
            agg_sh.at[pl.ds(_TAIL_ROW0, _TAIL_ROWS)],
        )

    plsc.subcore_barrier()

    def chunk_body(k, carry):
        g = wid + k * _NW

        @pl.when(g < _NCH)
        def _():
            pltpu.sync_copy(dst3.at[g], idx_d)
            pltpu.sync_copy(src3.at[g], idx_s)
            cps = [pltpu.make_async_copy(eg3.at[g], eg_in, sem)]
            for j in range(_IDXK):
                cps.append(pltpu.make_async_copy(
                    ug.at[idx_d.at[j]], bu.at[pl.ds(j * 128, 128)], sem))
                cps.append(pltpu.make_async_copy(
                    vg.at[idx_s.at[j]], bv.at[pl.ds(j * 128, 128)], sem))
            for cp in cps:
                cp.start()
            for cp in cps:
                cp.wait()

            def row_body(i, c2):
                for c8 in range(8):
                    r = i + (c8 // 4) * 128
                    sl = pl.ds((c8 % 4) * 16, 16)
                    s = (eg_in[i, pl.ds(c8 * 16, 16)]
                         + bu[r, sl] + bv[r, sl])
                    bu[r, sl] = jnp.maximum(s, 0.0)
                return c2

            lax.fori_loop(0, _C // 2, row_body, 0)

            for j in range(_IDXK):
                pltpu.sync_copy(
                    bu.at[pl.ds(j * 128, 128)],
                    agg_sh.at[idx_d.at[j]],
                    add=True,
                )

        return carry

    n_iter = (_NCH + _NW - 1) // _NW
    lax.fori_loop(0, n_iter, chunk_body, 0)

    plsc.subcore_barrier()
    pltpu.sync_copy(
        agg_sh.at[pl.ds(row0, _ROWS_PER_SUB)],
        out.at[cid, pl.ds(row0, _ROWS_PER_SUB)],
    )

    @pl.when(sid == 15)
    def _():
        pltpu.sync_copy(
            agg_sh.at[pl.ds(_TAIL_ROW0, _TAIL_ROWS)],
            out.at[cid, pl.ds(_TAIL_ROW0, _TAIL_ROWS)],
        )


@functools.cache
def _get_sc_edge():
    mesh = plsc.VectorSubcoreMesh(
        core_axis_name="c", subcore_axis_name="s", num_cores=2,
        num_subcores=16,
    )
    return pl.kernel(
        _sc_edge_body,
        out_type=jax.ShapeDtypeStruct((2, _U, _DG), jnp.float32),
        mesh=mesh,
        scratch_types=[
            pltpu.VMEM((_IDXK, 128), jnp.int32),
            pltpu.VMEM((_IDXK, 128), jnp.int32),
            pltpu.VMEM((_C // 2, 128), jnp.float32),
            pltpu.VMEM((_C, _DG), jnp.float32),
            pltpu.VMEM((_C, _DG), jnp.float32),
            pltpu.VMEM_SHARED((_U, _DG), jnp.float32),
            pltpu.SemaphoreType.DMA,
        ],
        compiler_params=pltpu.CompilerParams(use_tc_tiling_on_sc=False),
    )


@jax.jit
def _impl(u, v, e_indices, e_values, Wg, bg, Wf, bf):
    f_dim = u.shape[1]
    g_dim = v.shape[1]
    src = e_indices[0].astype(jnp.int32)
    dst = e_indices[1].astype(jnp.int32)

    ug_t = _mm(u, Wg[:f_dim], bg, 1000)
    vg_t = _mm(v, Wg[f_dim:f_dim + g_dim], jnp.zeros((_DG,), jnp.float32),
               1000)
    eg_br = 6400
    eg_t = _eg_mm(e_values, Wg[f_dim + g_dim:], eg_br)

    def _idx3(x):
        nb = _E // (2 * eg_br)
        return (x.reshape(nb, 2, eg_br // 128, 128)
                .swapaxes(1, 2).reshape(_NCH, _IDXK, 128))

    dst3 = _idx3(dst)
    src3 = _idx3(src)
    eg3 = eg_t.reshape(_NCH, _C // 2, 2 * _DG)
    zeros = jnp.zeros((_U, _DG), jnp.float32)

    agg2 = _get_sc_edge()(ug_t, vg_t, eg3, dst3, src3, zeros)

    return _f_mm(u, agg2[0], agg2[1], Wf[:f_dim], Wf[f_dim:], bf, 1000)


def kernel(u, v, e_indices, e_values, Wg, bg, Wf, bf):
    return _impl(u, v, e_indices, e_values, Wg, bg, Wf, bf)


# speedup vs baseline: 1.3682x; 1.3682x over previous
import functools

import jax
import jax.numpy as jnp
from jax import lax
from jax.experimental import pallas as pl
from jax.experimental.pallas import tpu as pltpu
from jax.experimental.pallas import tpu_sc as plsc

_U = 10000
_E = 320000
_DG = 64
_C = 256
_NCH = _E // _C
_NW = 32
_IDXK = _C // 128
_ROWS_PER_SUB = 624
_TAIL_ROW0 = 16 * _ROWS_PER_SUB
_TAIL_ROWS = _U - _TAIL_ROW0


def _mm_body(x_ref, w_ref, b_ref, o_ref):
    o_ref[...] = (
        jnp.dot(x_ref[...], w_ref[...], preferred_element_type=jnp.float32)
        + b_ref[...]
    )


def _mm(x, w, b, br):
    m, k = x.shape
    n = w.shape[1]
    return pl.pallas_call(
        _mm_body,
        grid=(m // br,),
        in_specs=[
            pl.BlockSpec((br, k), lambda i: (i, 0)),
            pl.BlockSpec((k, n), lambda i: (0, 0)),
            pl.BlockSpec((1, n), lambda i: (0, 0)),
        ],
        out_specs=pl.BlockSpec((br, n), lambda i: (i, 0)),
        out_shape=jax.ShapeDtypeStruct((m, n), jnp.float32),
    )(x, w, b.reshape(1, n))


def _eg_body(xa_ref, xb_ref, w_ref, o_ref):
    o_ref[...] = jnp.concatenate(
        [jnp.dot(xa_ref[...], w_ref[...], preferred_element_type=jnp.float32),
         jnp.dot(xb_ref[...], w_ref[...], preferred_element_type=jnp.float32)],
        axis=1,
    )


def _eg_mm(x, w, br):
    m2, k = x.shape
    m = m2 // 2
    return pl.pallas_call(
        _eg_body,
        grid=(m // br,),
        in_specs=[
            pl.BlockSpec((br, k), lambda i: (2 * i, 0)),
            pl.BlockSpec((br, k), lambda i: (2 * i + 1, 0)),
            pl.BlockSpec((k, _DG), lambda i: (0, 0)),
        ],
        out_specs=pl.BlockSpec((br, 2 * _DG), lambda i: (i, 0)),
        out_shape=jax.ShapeDtypeStruct((m, 2 * _DG), jnp.float32),
    )(x, x, w)


def _f_body(u_ref, a0_ref, a1_ref, wu_ref, wa_ref, b_ref, o_ref):
    acc = jnp.dot(u_ref[...], wu_ref[...], preferred_element_type=jnp.float32)
    acc = acc + jnp.dot(
        a0_ref[...] + a1_ref[...], wa_ref[...],
        preferred_element_type=jnp.float32,
    )
    o_ref[...] = jnp.maximum(acc + b_ref[...], 0.0)


def _f_mm(u, a0, a1, wu, wa, b, br):
    m, k = u.shape
    ka = a0.shape[1]
    n = wu.shape[1]
    return pl.pallas_call(
        _f_body,
        grid=(m // br,),
        in_specs=[
            pl.BlockSpec((br, k), lambda i: (i, 0)),
            pl.BlockSpec((br, ka), lambda i: (i, 0)),
            pl.BlockSpec((br, ka), lambda i: (i, 0)),
            pl.BlockSpec((k, n), lambda i: (0, 0)),
            pl.BlockSpec((ka, n), lambda i: (0, 0)),
            pl.BlockSpec((1, n), lambda i: (0, 0)),
        ],
        out_specs=pl.BlockSpec((br, n), lambda i: (i, 0)),
        out_shape=jax.ShapeDtypeStruct((m, n), jnp.float32),
    )(u, a0, a1, wu, wa, b.reshape(1, n))


def _sc_edge_body(ug, vg, eg3, dst3, src3, zeros_hbm, out, idx_d, idx_s,
                  eg_in, bu, bv, agg_sh, sem):
    cid = lax.axis_index("c")
    sid = lax.axis_index("s")
    wid = sid * 2 + cid

    row0 = sid * _ROWS_PER_SUB
    pltpu.sync_copy(
        zeros_hbm.at[pl.ds(row0, _ROWS_PER_SUB)],
        agg_sh.at[pl.ds(row0, _ROWS_PER_SUB)],
    )

    @pl.when(sid == 15)
    def _():
        pltpu.sync_copy(
            zeros_hbm.at[pl.ds(_TAIL_ROW0, _TAIL_ROWS)],
            agg_sh.at[pl.ds(_TAIL_ROW0, _TAIL_ROWS)],
        )

    plsc.subcore_barrier()

    def chunk_body(k, carry):
        g = wid + k * _NW

        @pl.when(g < _NCH)
        def _():
            pltpu.sync_copy(dst3.at[g], idx_d)
            pltpu.sync_copy(src3.at[g], idx_s)
            cps = [pltpu.make_async_copy(eg3.at[g], eg_in, sem)]
            for j in range(_IDXK):
                cps.append(pltpu.make_async_copy(
                    ug.at[idx_d.at[j]], bu.at[pl.ds(j * 128, 128)], sem))
                cps.append(pltpu.make_async_copy(
                    vg.at[idx_s.at[j]], bv.at[pl.ds(j * 128, 128)], sem))
            for cp in cps:
                cp.start()
            for cp in cps:
                cp.wait()

            @plsc.parallel_loop(0, _C // 2, unroll=4)
            def _(i):
                for c in range(4):
                    sl = pl.ds(c * 16, 16)
                    s = eg_in[i, sl] + bu[i, sl] + bv[i, sl]
                    bu[i, sl] = jnp.maximum(s, 0.0)

            @plsc.parallel_loop(0, _C // 2, unroll=4)
            def _(i):
                r = i + _C // 2
                for c in range(4):
                    sl = pl.ds(c * 16, 16)
                    s = (eg_in[i, pl.ds(64 + c * 16, 16)]
                         + bu[r, sl] + bv[r, sl])
                    bu[r, sl] = jnp.maximum(s, 0.0)

            for j in range(_IDXK):
                pltpu.sync_copy(
                    bu.at[pl.ds(j * 128, 128)],
                    agg_sh.at[idx_d.at[j]],
                    add=True,
                )

        return carry

    n_iter = (_NCH + _NW - 1) // _NW
    lax.fori_loop(0, n_iter, chunk_body, 0)

    plsc.subcore_barrier()
    pltpu.sync_copy(
        agg_sh.at[pl.ds(row0, _ROWS_PER_SUB)],
        out.at[cid, pl.ds(row0, _ROWS_PER_SUB)],
    )

    @pl.when(sid == 15)
    def _():
        pltpu.sync_copy(
            agg_sh.at[pl.ds(_TAIL_ROW0, _TAIL_ROWS)],
            out.at[cid, pl.ds(_TAIL_ROW0, _TAIL_ROWS)],
        )


@functools.cache
def _get_sc_edge():
    mesh = plsc.VectorSubcoreMesh(
        core_axis_name="c", subcore_axis_name="s", num_cores=2,
        num_subcores=16,
    )
    return pl.kernel(
        _sc_edge_body,
        out_type=jax.ShapeDtypeStruct((2, _U, _DG), jnp.float32),
        mesh=mesh,
        scratch_types=[
            pltpu.VMEM((_IDXK, 128), jnp.int32),
            pltpu.VMEM((_IDXK, 128), jnp.int32),
            pltpu.VMEM((_C // 2, 128), jnp.float32),
            pltpu.VMEM((_C, _DG), jnp.float32),
            pltpu.VMEM((_C, _DG), jnp.float32),
            pltpu.VMEM_SHARED((_U, _DG), jnp.float32),
            pltpu.SemaphoreType.DMA,
        ],
        compiler_params=pltpu.CompilerParams(use_tc_tiling_on_sc=False),
    )


@jax.jit
def _impl(u, v, e_indices, e_values, Wg, bg, Wf, bf):
    f_dim = u.shape[1]
    g_dim = v.shape[1]
    src = e_indices[0].astype(jnp.int32)
    dst = e_indices[1].astype(jnp.int32)

    ug_t = _mm(u, Wg[:f_dim], bg, 1000)
    vg_t = _mm(v, Wg[f_dim:f_dim + g_dim], jnp.zeros((_DG,), jnp.float32),
               1000)
    eg_br = 6400
    eg_t = _eg_mm(e_values, Wg[f_dim + g_dim:], eg_br)

    def _idx3(x):
        nb = _E // (2 * eg_br)
        return (x.reshape(nb, 2, eg_br // 128, 128)
                .swapaxes(1, 2).reshape(_NCH, _IDXK, 128))

    dst3 = _idx3(dst)
    src3 = _idx3(src)
    eg3 = eg_t.reshape(_NCH, _C // 2, 2 * _DG)
    zeros = jnp.zeros((_U, _DG), jnp.float32)

    agg2 = _get_sc_edge()(ug_t, vg_t, eg3, dst3, src3, zeros)

    return _f_mm(u, agg2[0], agg2[1], Wf[:f_dim], Wf[f_dim:], bf, 1000)


def kernel(u, v, e_indices, e_values, Wg, bg, Wf, bf):
    return _impl(u, v, e_indices, e_values, Wg, bg, Wf, bf)


# speedup vs baseline: 1.4560x; 1.0641x over previous
import functools

import jax
import jax.numpy as jnp
from jax import lax
from jax.experimental import pallas as pl
from jax.experimental.pallas import tpu as pltpu
from jax.experimental.pallas import tpu_sc as plsc

_U = 10000
_E = 320000
_DG = 64
_C = 128
_NCH = _E // _C
_NW = 32
_ROWS_PER_SUB = 624
_TAIL_ROW0 = 16 * _ROWS_PER_SUB
_TAIL_ROWS = _U - _TAIL_ROW0


def _mm_body(x_ref, w_ref, b_ref, o_ref):
    o_ref[...] = (
        jnp.dot(x_ref[...], w_ref[...], preferred_element_type=jnp.float32)
        + b_ref[...]
    )


def _mm(x, w, b, br):
    m, k = x.shape
    n = w.shape[1]
    return pl.pallas_call(
        _mm_body,
        grid=(m // br,),
        in_specs=[
            pl.BlockSpec((br, k), lambda i: (i, 0)),
            pl.BlockSpec((k, n), lambda i: (0, 0)),
            pl.BlockSpec((1, n), lambda i: (0, 0)),
        ],
        out_specs=pl.BlockSpec((br, n), lambda i: (i, 0)),
        out_shape=jax.ShapeDtypeStruct((m, n), jnp.float32),
    )(x, w, b.reshape(1, n))


def _eg_body(xa_ref, xb_ref, w_ref, o_ref):
    o_ref[...] = jnp.concatenate(
        [jnp.dot(xa_ref[...], w_ref[...], preferred_element_type=jnp.float32),
         jnp.dot(xb_ref[...], w_ref[...], preferred_element_type=jnp.float32)],
        axis=1,
    )


def _eg_mm(x, w, br):
    m2, k = x.shape
    m = m2 // 2
    return pl.pallas_call(
        _eg_body,
        grid=(m // br,),
        in_specs=[
            pl.BlockSpec((br, k), lambda i: (2 * i, 0)),
            pl.BlockSpec((br, k), lambda i: (2 * i + 1, 0)),
            pl.BlockSpec((k, _DG), lambda i: (0, 0)),
        ],
        out_specs=pl.BlockSpec((br, 2 * _DG), lambda i: (i, 0)),
        out_shape=jax.ShapeDtypeStruct((m, 2 * _DG), jnp.float32),
    )(x, x, w)


def _f_body(u_ref, a0_ref, a1_ref, wu_ref, wa_ref, b_ref, o_ref):
    acc = jnp.dot(u_ref[...], wu_ref[...], preferred_element_type=jnp.float32)
    acc = acc + jnp.dot(
        a0_ref[...] + a1_ref[...], wa_ref[...],
        preferred_element_type=jnp.float32,
    )
    o_ref[...] = jnp.maximum(acc + b_ref[...], 0.0)


def _f_mm(u, a0, a1, wu, wa, b, br):
    m, k = u.shape
    ka = a0.shape[1]
    n = wu.shape[1]
    return pl.pallas_call(
        _f_body,
        grid=(m // br,),
        in_specs=[
            pl.BlockSpec((br, k), lambda i: (i, 0)),
            pl.BlockSpec((br, ka), lambda i: (i, 0)),
            pl.BlockSpec((br, ka), lambda i: (i, 0)),
            pl.BlockSpec((k, n), lambda i: (0, 0)),
            pl.BlockSpec((ka, n), lambda i: (0, 0)),
            pl.BlockSpec((1, n), lambda i: (0, 0)),
        ],
        out_specs=pl.BlockSpec((br, n), lambda i: (i, 0)),
        out_shape=jax.ShapeDtypeStruct((m, n), jnp.float32),
    )(u, a0, a1, wu, wa, b.reshape(1, n))


def _sc_edge_body(ug, vg, eg3, dst3, src3, zeros_hbm, out,
                  idd0, ids0, eg0, bu0, bv0, sg0, ss0,
                  idd1, ids1, eg1, bu1, bv1, sg1, ss1, agg_sh):
    cid = lax.axis_index("c")
    sid = lax.axis_index("s")
    wid = sid * 2 + cid

    row0 = sid * _ROWS_PER_SUB
    pltpu.sync_copy(
        zeros_hbm.at[pl.ds(row0, _ROWS_PER_SUB)],
        agg_sh.at[pl.ds(row0, _ROWS_PER_SUB)],
    )

    @pl.when(sid == 15)
    def _():
        pltpu.sync_copy(
            zeros_hbm.at[pl.ds(_TAIL_ROW0, _TAIL_ROWS)],
            agg_sh.at[pl.ds(_TAIL_ROW0, _TAIL_ROWS)],
        )

    plsc.subcore_barrier()

    slots = ((idd0, ids0, eg0, bu0, bv0, sg0, ss0),
             (idd1, ids1, eg1, bu1, bv1, sg1, ss1))
    half = _C // 2

    def gather_cps(g, sl):
        idd, ids, eg_in, bu, bv, sg, _ = sl
        return [
            pltpu.make_async_copy(eg3.at[g], eg_in, sg),
            pltpu.make_async_copy(ug.at[idd.at[0]], bu.at[pl.ds(0, half)],
                                  sg),
            pltpu.make_async_copy(ug.at[idd.at[1]], bu.at[pl.ds(half, half)],
                                  sg),
            pltpu.make_async_copy(vg.at[ids.at[0]], bv.at[pl.ds(0, half)],
                                  sg),
            pltpu.make_async_copy(vg.at[ids.at[1]], bv.at[pl.ds(half, half)],
                                  sg),
        ]

    def start_chunk(g, sl):
        pltpu.sync_copy(dst3.at[g], sl[0])
        pltpu.sync_copy(src3.at[g], sl[1])
        for cp in gather_cps(g, sl):
            cp.start()

    def scatter_cps(sl):
        idd, _, _, bu, _, _, ss = sl
        return [
            pltpu.make_async_copy(bu.at[pl.ds(0, half)], agg_sh.at[idd.at[0]],
                                  ss),
            pltpu.make_async_copy(bu.at[pl.ds(half, half)],
                                  agg_sh.at[idd.at[1]], ss),
        ]

    def compute(sl):
        _, _, eg_in, bu, bv, _, _ = sl

        @plsc.parallel_loop(0, half, unroll=4)
        def _(i):
            for c in range(4):
                sl16 = pl.ds(c * 16, 16)
                s = eg_in[i, sl16] + bu[i, sl16] + bv[i, sl16]
                bu[i, sl16] = jnp.maximum(s, 0.0)

        @plsc.parallel_loop(0, half, unroll=4)
        def _(i):
            r = i + half
            for c in range(4):
                sl16 = pl.ds(c * 16, 16)
                s = (eg_in[i, pl.ds(64 + c * 16, 16)]
                     + bu[r, sl16] + bv[r, sl16])
                bu[r, sl16] = jnp.maximum(s, 0.0)

    start_chunk(wid, slots[0])

    def pair_body(k2, carry):
        for off in (0, 1):
            k = 2 * k2 + off
            s, s2 = off, 1 - off
            g = wid + k * _NW
            gn = g + _NW

            @pl.when(g < _NCH)
            def _():
                for cp in gather_cps(g, slots[s]):
                    cp.wait()

            @pl.when((gn < _NCH) & (k >= 1))
            def _():
                for cp in scatter_cps(slots[s2]):
                    cp.wait()

            @pl.when(gn < _NCH)
            def _():
                start_chunk(gn, slots[s2])

            @pl.when(g < _NCH)
            def _():
                compute(slots[s])
                for cp in scatter_cps(slots[s]):
                    cp.start(add=True)

        return carry

    n_pairs = (_NCH // _NW + 2) // 2
    lax.fori_loop(0, n_pairs, pair_body, 0)

    for s in (0, 1):
        for cp in scatter_cps(slots[s]):
            cp.wait()

    plsc.subcore_barrier()
    pltpu.sync_copy(
        agg_sh.at[pl.ds(row0, _ROWS_PER_SUB)],
        out.at[cid, pl.ds(row0, _ROWS_PER_SUB)],
    )

    @pl.when(sid == 15)
    def _():
        pltpu.sync_copy(
            agg_sh.at[pl.ds(_TAIL_ROW0, _TAIL_ROWS)],
            out.at[cid, pl.ds(_TAIL_ROW0, _TAIL_ROWS)],
        )


@functools.cache
def _get_sc_edge():
    mesh = plsc.VectorSubcoreMesh(
        core_axis_name="c", subcore_axis_name="s", num_cores=2,
        num_subcores=16,
    )
    return pl.kernel(
        _sc_edge_body,
        out_type=jax.ShapeDtypeStruct((2, _U, _DG), jnp.float32),
        mesh=mesh,
        scratch_types=(
            2 * [
                pltpu.VMEM((2, _C // 2), jnp.int32),
                pltpu.VMEM((2, _C // 2), jnp.int32),
                pltpu.VMEM((_C // 2, 128), jnp.float32),
                pltpu.VMEM((_C, _DG), jnp.float32),
                pltpu.VMEM((_C, _DG), jnp.float32),
                pltpu.SemaphoreType.DMA,
                pltpu.SemaphoreType.DMA,
            ]
            + [pltpu.VMEM_SHARED((_U, _DG), jnp.float32)]
        ),
        compiler_params=pltpu.CompilerParams(use_tc_tiling_on_sc=False),
    )


@jax.jit
def _impl(u, v, e_indices, e_values, Wg, bg, Wf, bf):
    f_dim = u.shape[1]
    g_dim = v.shape[1]
    src = e_indices[0].astype(jnp.int32)
    dst = e_indices[1].astype(jnp.int32)

    ug_t = _mm(u, Wg[:f_dim], bg, 1000)
    vg_t = _mm(v, Wg[f_dim:f_dim + g_dim], jnp.zeros((_DG,), jnp.float32),
               1000)
    eg_br = 6400
    eg_t = _eg_mm(e_values, Wg[f_dim + g_dim:], eg_br)

    half = _C // 2

    def _idx3(x):
        nb = _E // (2 * eg_br)
        return (x.reshape(nb, 2, eg_br // half, half)
                .swapaxes(1, 2).reshape(_NCH, 2, half))

    dst3 = _idx3(dst)
    src3 = _idx3(src)
    eg3 = eg_t.reshape(_NCH, half, 2 * _DG)
    zeros = jnp.zeros((_U, _DG), jnp.float32)

    agg2 = _get_sc_edge()(ug_t, vg_t, eg3, dst3, src3, zeros)

    return _f_mm(u, agg2[0], agg2[1], Wf[:f_dim], Wf[f_dim:], bf, 1000)


def kernel(u, v, e_indices, e_values, Wg, bg, Wf, bf):
    return _impl(u, v, e_indices, e_values, Wg, bg, Wf, bf)


# speedup vs baseline: 1.4853x; 1.0202x over previous
import functools

import jax
import jax.numpy as jnp
from jax import lax
from jax.experimental import pallas as pl
from jax.experimental.pallas import tpu as pltpu
from jax.experimental.pallas import tpu_sc as plsc

_U = 10000
_E = 320000
_DG = 64
_C = 128
_NCH = _E // _C
_NW = 32
_ROWS_PER_SUB = 624
_TAIL_ROW0 = 16 * _ROWS_PER_SUB
_TAIL_ROWS = _U - _TAIL_ROW0


def _mm_body(x_ref, w_ref, b_ref, o_ref):
    o_ref[...] = (
        jnp.dot(x_ref[...], w_ref[...], preferred_element_type=jnp.float32)
        + b_ref[...]
    )


def _mm(x, w, b, br):
    m, k = x.shape
    n = w.shape[1]
    return pl.pallas_call(
        _mm_body,
        grid=(m // br,),
        in_specs=[
            pl.BlockSpec((br, k), lambda i: (i, 0)),
            pl.BlockSpec((k, n), lambda i: (0, 0)),
            pl.BlockSpec((1, n), lambda i: (0, 0)),
        ],
        out_specs=pl.BlockSpec((br, n), lambda i: (i, 0)),
        out_shape=jax.ShapeDtypeStruct((m, n), jnp.float32),
    )(x, w, b.reshape(1, n))


def _eg_body(xa_ref, xb_ref, w_ref, o_ref):
    o_ref[...] = jnp.concatenate(
        [jnp.dot(xa_ref[...], w_ref[...], preferred_element_type=jnp.float32),
         jnp.dot(xb_ref[...], w_ref[...], preferred_element_type=jnp.float32)],
        axis=1,
    )


def _eg_mm(x, w, br):
    m2, k = x.shape
    m = m2 // 2
    return pl.pallas_call(
        _eg_body,
        grid=(m // br,),
        in_specs=[
            pl.BlockSpec((br, k), lambda i: (2 * i, 0)),
            pl.BlockSpec((br, k), lambda i: (2 * i + 1, 0)),
            pl.BlockSpec((k, _DG), lambda i: (0, 0)),
        ],
        out_specs=pl.BlockSpec((br, 2 * _DG), lambda i: (i, 0)),
        out_shape=jax.ShapeDtypeStruct((m, 2 * _DG), jnp.float32),
    )(x, x, w)


def _f_body(u_ref, a0_ref, a1_ref, a2_ref, a3_ref, wu_ref, wa_ref, b_ref,
            o_ref):
    acc = jnp.dot(u_ref[...], wu_ref[...], preferred_element_type=jnp.float32)
    agg = (a0_ref[...] + a1_ref[...]) + (a2_ref[...] + a3_ref[...])
    acc = acc + jnp.dot(agg, wa_ref[...], preferred_element_type=jnp.float32)
    o_ref[...] = jnp.maximum(acc + b_ref[...], 0.0)


def _f_mm(u, a4, wu, wa, b, br):
    m, k = u.shape
    ka = a4[0].shape[1]
    n = wu.shape[1]
    row_spec = pl.BlockSpec((br, ka), lambda i: (i, 0))
    return pl.pallas_call(
        _f_body,
        grid=(m // br,),
        in_specs=[
            pl.BlockSpec((br, k), lambda i: (i, 0)),
            row_spec, row_spec, row_spec, row_spec,
            pl.BlockSpec((k, n), lambda i: (0, 0)),
            pl.BlockSpec((ka, n), lambda i: (0, 0)),
            pl.BlockSpec((1, n), lambda i: (0, 0)),
        ],
        out_specs=pl.BlockSpec((br, n), lambda i: (i, 0)),
        out_shape=jax.ShapeDtypeStruct((m, n), jnp.float32),
    )(u, *a4, wu, wa, b.reshape(1, n))


def _sc_edge_body(nch, ug, vg, eg3, dst3, src3, zeros_hbm, out,
                  idd0, ids0, eg0, bu0, bv0, sg0, ss0,
                  idd1, ids1, eg1, bu1, bv1, sg1, ss1, agg_sh):
    cid = lax.axis_index("c")
    sid = lax.axis_index("s")
    wid = sid * 2 + cid

    row0 = sid * _ROWS_PER_SUB
    pltpu.sync_copy(
        zeros_hbm.at[pl.ds(row0, _ROWS_PER_SUB)],
        agg_sh.at[pl.ds(row0, _ROWS_PER_SUB)],
    )

    @pl.when(sid == 15)
    def _():
        pltpu.sync_copy(
            zeros_hbm.at[pl.ds(_TAIL_ROW0, _TAIL_ROWS)],
            agg_sh.at[pl.ds(_TAIL_ROW0, _TAIL_ROWS)],
        )

    plsc.subcore_barrier()

    slots = ((idd0, ids0, eg0, bu0, bv0, sg0, ss0),
             (idd1, ids1, eg1, bu1, bv1, sg1, ss1))
    half = _C // 2

    def gather_cps(g, sl):
        idd, ids, eg_in, bu, bv, sg, _ = sl
        return [
            pltpu.make_async_copy(eg3.at[g], eg_in, sg),
            pltpu.make_async_copy(ug.at[idd.at[0]], bu.at[pl.ds(0, half)],
                                  sg),
            pltpu.make_async_copy(ug.at[idd.at[1]], bu.at[pl.ds(half, half)],
                                  sg),
            pltpu.make_async_copy(vg.at[ids.at[0]], bv.at[pl.ds(0, half)],
                                  sg),
            pltpu.make_async_copy(vg.at[ids.at[1]], bv.at[pl.ds(half, half)],
                                  sg),
        ]

    def start_chunk(g, sl):
        pltpu.sync_copy(dst3.at[g], sl[0])
        pltpu.sync_copy(src3.at[g], sl[1])
        for cp in gather_cps(g, sl):
            cp.start()

    def scatter_cps(sl):
        idd, _, _, bu, _, _, ss = sl
        return [
            pltpu.make_async_copy(bu.at[pl.ds(0, half)], agg_sh.at[idd.at[0]],
                                  ss),
            pltpu.make_async_copy(bu.at[pl.ds(half, half)],
                                  agg_sh.at[idd.at[1]], ss),
        ]

    def compute(sl):
        _, _, eg_in, bu, bv, _, _ = sl

        @plsc.parallel_loop(0, half, unroll=4)
        def _(i):
            for c in range(4):
                sl16 = pl.ds(c * 16, 16)
                s = eg_in[i, sl16] + bu[i, sl16] + bv[i, sl16]
                bu[i, sl16] = jnp.maximum(s, 0.0)

        @plsc.parallel_loop(0, half, unroll=4)
        def _(i):
            r = i + half
            for c in range(4):
                sl16 = pl.ds(c * 16, 16)
                s = (eg_in[i, pl.ds(64 + c * 16, 16)]
                     + bu[r, sl16] + bv[r, sl16])
                bu[r, sl16] = jnp.maximum(s, 0.0)

    start_chunk(wid, slots[0])

    def pair_body(k2, carry):
        for off in (0, 1):
            k = 2 * k2 + off
            s, s2 = off, 1 - off
            g = wid + k * _NW
            gn = g + _NW

            @pl.when(g < nch)
            def _():
                for cp in gather_cps(g, slots[s]):
                    cp.wait()

            @pl.when((gn < nch) & (k >= 1))
            def _():
                for cp in scatter_cps(slots[s2]):
                    cp.wait()

            @pl.when(gn < nch)
            def _():
                start_chunk(gn, slots[s2])

            @pl.when(g < nch)
            def _():
                compute(slots[s])
                for cp in scatter_cps(slots[s]):
                    cp.start(add=True)

        return carry

    n_pairs = (nch // _NW + 2) // 2
    lax.fori_loop(0, n_pairs, pair_body, 0)

    for s in (0, 1):
        for cp in scatter_cps(slots[s]):
            cp.wait()

    plsc.subcore_barrier()
    pltpu.sync_copy(
        agg_sh.at[pl.ds(row0, _ROWS_PER_SUB)],
        out.at[cid, pl.ds(row0, _ROWS_PER_SUB)],
    )

    @pl.when(sid == 15)
    def _():
        pltpu.sync_copy(
            agg_sh.at[pl.ds(_TAIL_ROW0, _TAIL_ROWS)],
            out.at[cid, pl.ds(_TAIL_ROW0, _TAIL_ROWS)],
        )


@functools.cache
def _get_sc_edge(nch):
    mesh = plsc.VectorSubcoreMesh(
        core_axis_name="c", subcore_axis_name="s", num_cores=2,
        num_subcores=16,
    )
    return pl.kernel(
        functools.partial(_sc_edge_body, nch),
        out_type=jax.ShapeDtypeStruct((2, _U, _DG), jnp.float32),
        mesh=mesh,
        scratch_types=(
            2 * [
                pltpu.VMEM((2, _C // 2), jnp.int32),
                pltpu.VMEM((2, _C // 2), jnp.int32),
                pltpu.VMEM((_C // 2, 128), jnp.float32),
                pltpu.VMEM((_C, _DG), jnp.float32),
                pltpu.VMEM((_C, _DG), jnp.float32),
                pltpu.SemaphoreType.DMA,
                pltpu.SemaphoreType.DMA,
            ]
            + [pltpu.VMEM_SHARED((_U, _DG), jnp.float32)]
        ),
        compiler_params=pltpu.CompilerParams(use_tc_tiling_on_sc=False),
    )


@jax.jit
def _impl(u, v, e_indices, e_values, Wg, bg, Wf, bf):
    f_dim = u.shape[1]
    g_dim = v.shape[1]
    src = e_indices[0].astype(jnp.int32)
    dst = e_indices[1].astype(jnp.int32)

    ug_t = _mm(u, Wg[:f_dim], bg, 1000)
    vg_t = _mm(v, Wg[f_dim:f_dim + g_dim], jnp.zeros((_DG,), jnp.float32),
               1000)
    zeros = jnp.zeros((_U, _DG), jnp.float32)

    half = _C // 2
    eg_br = 8000
    e_half = _E // 2
    nch_h = e_half // _C
    aggs = []
    for h in range(2):
        sl = slice(h * e_half, (h + 1) * e_half)
        eg_t = _eg_mm(e_values[sl], Wg[f_dim + g_dim:], eg_br)

        def _idx3(x):
            nb = e_half // (2 * eg_br)
            return (x[sl].reshape(nb, 2, eg_br // half, half)
                    .swapaxes(1, 2).reshape(nch_h, 2, half))

        agg2 = _get_sc_edge(nch_h)(
            ug_t, vg_t, eg_t.reshape(nch_h, half, 2 * _DG),
            _idx3(dst), _idx3(src), zeros)
        aggs.extend([agg2[0], agg2[1]])

    return _f_mm(u, aggs, Wf[:f_dim], Wf[f_dim:], bf, 1000)


def kernel(u, v, e_indices, e_values, Wg, bg, Wf, bf):
    return _impl(u, v, e_indices, e_values, Wg, bg, Wf, bf)


# speedup vs baseline: 1.5520x; 1.0449x over previous
import functools

import jax
import jax.numpy as jnp
from jax import lax
from jax.experimental import pallas as pl
from jax.experimental.pallas import tpu as pltpu
from jax.experimental.pallas import tpu_sc as plsc

_U = 10000
_E = 320000
_DG = 64
_C = 128
_NCH = _E // _C
_NW = 32
_ROWS_PER_SUB = 624
_TAIL_ROW0 = 16 * _ROWS_PER_SUB
_TAIL_ROWS = _U - _TAIL_ROW0


def _uv_body(u_ref, v_ref, wu_ref, wv_ref, b_ref, ug_ref, vg_ref):
    ug_ref[...] = (
        jnp.dot(u_ref[...], wu_ref[...], preferred_element_type=jnp.float32)
        + b_ref[...]
    )
    vg_ref[...] = jnp.dot(v_ref[...], wv_ref[...],
                          preferred_element_type=jnp.float32)


def _uv_mm(u, v, wu, wv, b, br):
    m, k = u.shape
    n = wu.shape[1]
    out = jax.ShapeDtypeStruct((m, n), jnp.float32)
    return pl.pallas_call(
        _uv_body,
        grid=(m // br,),
        in_specs=[
            pl.BlockSpec((br, k), lambda i: (i, 0)),
            pl.BlockSpec((br, k), lambda i: (i, 0)),
            pl.BlockSpec((k, n), lambda i: (0, 0)),
            pl.BlockSpec((k, n), lambda i: (0, 0)),
            pl.BlockSpec((1, n), lambda i: (0, 0)),
        ],
        out_specs=[pl.BlockSpec((br, n), lambda i: (i, 0))] * 2,
        out_shape=[out, out],
    )(u, v, wu, wv, b.reshape(1, n))


def _eg_body(xa_ref, xb_ref, w_ref, o_ref):
    o_ref[...] = jnp.concatenate(
        [jnp.dot(xa_ref[...], w_ref[...], preferred_element_type=jnp.float32),
         jnp.dot(xb_ref[...], w_ref[...], preferred_element_type=jnp.float32)],
        axis=1,
    )


def _eg_mm(x, w, br):
    m2, k = x.shape
    m = m2 // 2
    return pl.pallas_call(
        _eg_body,
        grid=(m // br,),
        in_specs=[
            pl.BlockSpec((br, k), lambda i: (2 * i, 0)),
            pl.BlockSpec((br, k), lambda i: (2 * i + 1, 0)),
            pl.BlockSpec((k, _DG), lambda i: (0, 0)),
        ],
        out_specs=pl.BlockSpec((br, 2 * _DG), lambda i: (i, 0)),
        out_shape=jax.ShapeDtypeStruct((m, 2 * _DG), jnp.float32),
    )(x, x, w)


def _f_body(u_ref, a0_ref, a1_ref, a2_ref, a3_ref, wu_ref, wa_ref, b_ref,
            o_ref):
    acc = jnp.dot(u_ref[...], wu_ref[...], preferred_element_type=jnp.float32)
    agg = (a0_ref[...] + a1_ref[...]) + (a2_ref[...] + a3_ref[...])
    acc = acc + jnp.dot(agg, wa_ref[...], preferred_element_type=jnp.float32)
    o_ref[...] = jnp.maximum(acc + b_ref[...], 0.0)


def _f_mm(u, a4, wu, wa, b, br):
    m, k = u.shape
    ka = a4[0].shape[1]
    n = wu.shape[1]
    row_spec = pl.BlockSpec((br, ka), lambda i: (i, 0))
    return pl.pallas_call(
        _f_body,
        grid=(m // br,),
        in_specs=[
            pl.BlockSpec((br, k), lambda i: (i, 0)),
            row_spec, row_spec, row_spec, row_spec,
            pl.BlockSpec((k, n), lambda i: (0, 0)),
            pl.BlockSpec((ka, n), lambda i: (0, 0)),
            pl.BlockSpec((1, n), lambda i: (0, 0)),
        ],
        out_specs=pl.BlockSpec((br, n), lambda i: (i, 0)),
        out_shape=jax.ShapeDtypeStruct((m, n), jnp.float32),
    )(u, *a4, wu, wa, b.reshape(1, n))


def _sc_edge_body(nch, ug, vg, eg3, dst3, src3, zeros_hbm, out,
                  idd0, ids0, eg0, bu0, bv0, sg0, ss0,
                  idd1, ids1, eg1, bu1, bv1, sg1, ss1, agg_sh):
    cid = lax.axis_index("c")
    sid = lax.axis_index("s")
    wid = sid * 2 + cid

    row0 = sid * _ROWS_PER_SUB
    pltpu.sync_copy(
        zeros_hbm.at[pl.ds(row0, _ROWS_PER_SUB)],
        agg_sh.at[pl.ds(row0, _ROWS_PER_SUB)],
    )

    @pl.when(sid == 15)
    def _():
        pltpu.sync_copy(
            zeros_hbm.at[pl.ds(_TAIL_ROW0, _TAIL_ROWS)],
            agg_sh.at[pl.ds(_TAIL_ROW0, _TAIL_ROWS)],
        )

    plsc.subcore_barrier()

    slots = ((idd0, ids0, eg0, bu0, bv0, sg0, ss0),
             (idd1, ids1, eg1, bu1, bv1, sg1, ss1))
    half = _C // 2

    def gather_cps(g, sl):
        idd, ids, eg_in, bu, bv, sg, _ = sl
        return [
            pltpu.make_async_copy(eg3.at[g], eg_in, sg),
            pltpu.make_async_copy(ug.at[idd.at[0]], bu.at[pl.ds(0, half)],
                                  sg),
            pltpu.make_async_copy(ug.at[idd.at[1]], bu.at[pl.ds(half, half)],
                                  sg),
            pltpu.make_async_copy(vg.at[ids.at[0]], bv.at[pl.ds(0, half)],
                                  sg),
            pltpu.make_async_copy(vg.at[ids.at[1]], bv.at[pl.ds(half, half)],
                                  sg),
        ]

    def start_chunk(g, sl):
        pltpu.sync_copy(dst3.at[g], sl[0])
        pltpu.sync_copy(src3.at[g], sl[1])
        for cp in gather_cps(g, sl):
            cp.start()

    def scatter_cps(sl):
        idd, _, _, bu, _, _, ss = sl
        return [
            pltpu.make_async_copy(bu.at[pl.ds(0, half)], agg_sh.at[idd.at[0]],
                                  ss),
            pltpu.make_async_copy(bu.at[pl.ds(half, half)],
                                  agg_sh.at[idd.at[1]], ss),
        ]

    def compute(sl):
        _, _, eg_in, bu, bv, _, _ = sl

        @plsc.parallel_loop(0, half, unroll=4)
        def _(i):
            for c in range(4):
                sl16 = pl.ds(c * 16, 16)
                s = eg_in[i, sl16] + bu[i, sl16] + bv[i, sl16]
                bu[i, sl16] = jnp.maximum(s, 0.0)

        @plsc.parallel_loop(0, half, unroll=4)
        def _(i):
            r = i + half
            for c in range(4):
                sl16 = pl.ds(c * 16, 16)
                s = (eg_in[i, pl.ds(64 + c * 16, 16)]
                     + bu[r, sl16] + bv[r, sl16])
                bu[r, sl16] = jnp.maximum(s, 0.0)

    start_chunk(wid, slots[0])

    def pair_body(k2, carry):
        for off in (0, 1):
            k = 2 * k2 + off
            s, s2 = off, 1 - off
            g = wid + k * _NW
            gn = g + _NW

            @pl.when(g < nch)
            def _():
                for cp in gather_cps(g, slots[s]):
                    cp.wait()

            @pl.when((gn < nch) & (k >= 1))
            def _():
                for cp in scatter_cps(slots[s2]):
                    cp.wait()

            @pl.when(gn < nch)
            def _():
                start_chunk(gn, slots[s2])

            @pl.when(g < nch)
            def _():
                compute(slots[s])
                for cp in scatter_cps(slots[s]):
                    cp.start(add=True)

        return carry

    n_pairs = (nch // _NW + 2) // 2
    lax.fori_loop(0, n_pairs, pair_body, 0)

    for s in (0, 1):
        for cp in scatter_cps(slots[s]):
            cp.wait()

    plsc.subcore_barrier()
    pltpu.sync_copy(
        agg_sh.at[pl.ds(row0, _ROWS_PER_SUB)],
        out.at[cid, pl.ds(row0, _ROWS_PER_SUB)],
    )

    @pl.when(sid == 15)
    def _():
        pltpu.sync_copy(
            agg_sh.at[pl.ds(_TAIL_ROW0, _TAIL_ROWS)],
            out.at[cid, pl.ds(_TAIL_ROW0, _TAIL_ROWS)],
        )


@functools.cache
def _get_sc_edge(nch):
    mesh = plsc.VectorSubcoreMesh(
        core_axis_name="c", subcore_axis_name="s", num_cores=2,
        num_subcores=16,
    )
    return pl.kernel(
        functools.partial(_sc_edge_body, nch),
        out_type=jax.ShapeDtypeStruct((2, _U, _DG), jnp.float32),
        mesh=mesh,
        scratch_types=(
            2 * [
                pltpu.VMEM((2, _C // 2), jnp.int32),
                pltpu.VMEM((2, _C // 2), jnp.int32),
                pltpu.VMEM((_C // 2, 128), jnp.float32),
                pltpu.VMEM((_C, _DG), jnp.float32),
                pltpu.VMEM((_C, _DG), jnp.float32),
                pltpu.SemaphoreType.DMA,
                pltpu.SemaphoreType.DMA,
            ]
            + [pltpu.VMEM_SHARED((_U, _DG), jnp.float32)]
        ),
        compiler_params=pltpu.CompilerParams(use_tc_tiling_on_sc=False),
    )


@jax.jit
def _impl(u, v, e_indices, e_values, Wg, bg, Wf, bf):
    f_dim = u.shape[1]
    g_dim = v.shape[1]
    src = e_indices[0].astype(jnp.int32)
    dst = e_indices[1].astype(jnp.int32)

    ug_t, vg_t = _uv_mm(u, v, Wg[:f_dim], Wg[f_dim:f_dim + g_dim], bg, 2000)
    zeros = jnp.zeros((_U, _DG), jnp.float32)

    half = _C // 2
    eg_br = 8000
    e_half = _E // 2
    nch_h = e_half // _C

    def _idx3(x, sl):
        nb = e_half // (2 * eg_br)
        return (x[sl].reshape(nb, 2, eg_br // half, half)
                .swapaxes(1, 2).reshape(nch_h, 2, half))

    halves = []
    for h in range(2):
        sl = slice(h * e_half, (h + 1) * e_half)
        eg_t = _eg_mm(e_values[sl], Wg[f_dim + g_dim:], eg_br)
        halves.append((eg_t, _idx3(dst, sl), _idx3(src, sl)))

    aggs = []
    for eg_t, dst3, src3 in halves:
        agg2 = _get_sc_edge(nch_h)(
            ug_t, vg_t, eg_t.reshape(nch_h, half, 2 * _DG), dst3, src3,
            zeros)
        aggs.extend([agg2[0], agg2[1]])

    return _f_mm(u, aggs, Wf[:f_dim], Wf[f_dim:], bf, 1000)


def kernel(u, v, e_indices, e_values, Wg, bg, Wf, bf):
    return _impl(u, v, e_indices, e_values, Wg, bg, Wf, bf)
